# baseline (speedup 1.0000x reference)
import functools

import jax
import jax.numpy as jnp
from jax import lax
from jax.experimental import pallas as pl
from jax.experimental.pallas import tpu as pltpu
from jax.experimental.pallas import tpu_sc as plsc

N = 10000
E = 320000
D = 128
F32 = jnp.float32

NC = 2
NS = 16
EPC = E // NC
EPW = EPC // NS
CH = 16
NCHUNK = EPW // CH
NTRI = 208
RCH = 16
NRCH = N // RCH


def _dot(a, b):
    return jnp.dot(a, b, preferred_element_type=F32)


def _projc_body(x_ref, w1_ref, w2_ref, o1_ref, o2_ref):
    x = x_ref[...]
    o1_ref[...] = _dot(x, w1_ref[...])
    o2_ref[...] = _dot(x, w2_ref[...])


def _projc(x, w1, w2, blk):
    rows, k = x.shape
    grid = rows // blk
    ospec = pl.BlockSpec((blk, D), lambda i: (i, 0))
    return pl.pallas_call(
        _projc_body,
        grid=(grid,),
        in_specs=[
            pl.BlockSpec((blk, k), lambda i: (i, 0)),
            pl.BlockSpec((k, D), lambda i: (0, 0)),
            pl.BlockSpec((k, D), lambda i: (0, 0)),
        ],
        out_specs=[ospec, ospec],
        out_shape=[
            jax.ShapeDtypeStruct((rows, D), F32),
            jax.ShapeDtypeStruct((rows, D), F32),
        ],
    )(x, w1, w2)


def _projab_body(x_ref, w_ref, o_ref):
    o_ref[...] = _dot(x_ref[...], w_ref[...])


def _projab(x, w, blk=2000):
    grid = N // blk
    return pl.pallas_call(
        _projab_body,
        grid=(grid,),
        in_specs=[
            pl.BlockSpec((blk, D), lambda i: (i, 0)),
            pl.BlockSpec((D, 2 * D), lambda i: (0, 0)),
        ],
        out_specs=pl.BlockSpec((blk, 2 * D), lambda i: (i, 0)),
        out_shape=jax.ShapeDtypeStruct((N, 2 * D), F32),
    )(x, w)


def _red(sa_ref, sb_ref, we2_ref):
    return _dot(sa_ref[...] + sb_ref[...], we2_ref[...])


def _node_body(sa_ref, sb_ref, x_ref, we2_ref, wn1a_ref, wn1b_ref, wn2_ref,
               wab_ref, h_ref, ab_ref):
    red = _red(sa_ref, sb_ref, we2_ref)
    z = _dot(x_ref[...], wn1a_ref[...]) + _dot(red, wn1b_ref[...])
    h = _dot(jnp.maximum(z, 0.01 * z), wn2_ref[...])
    h_ref[...] = h
    ab_ref[...] = _dot(h, wab_ref[...])


def _node(sa, sb, x, we2, wn1a, wn1b, wn2, wab, blk=2000):
    grid = N // blk
    wspec = pl.BlockSpec((D, D), lambda i: (0, 0))
    rspec = pl.BlockSpec((blk, D), lambda i: (i, 0))
    return pl.pallas_call(
        _node_body,
        grid=(grid,),
        in_specs=[rspec, rspec, rspec] + [wspec] * 4
        + [pl.BlockSpec((D, 2 * D), lambda i: (0, 0))],
        out_specs=[rspec, pl.BlockSpec((blk, 2 * D), lambda i: (i, 0))],
        out_shape=[jax.ShapeDtypeStruct((N, D), F32),
                   jax.ShapeDtypeStruct((N, 2 * D), F32)],
    )(sa, sb, x, we2, wn1a, wn1b, wn2, wab)


def _final_body(sa_ref, sb_ref, x_ref, we2_ref, wn1a_ref, wn1b_ref, wn2_ref,
                o_ref):
    red = _red(sa_ref, sb_ref, we2_ref)
    z = _dot(x_ref[...], wn1a_ref[...]) + _dot(red, wn1b_ref[...])
    o_ref[...] = _dot(jnp.maximum(z, 0.01 * z), wn2_ref[...])


def _final(sa, sb, x, we2, wn1a, wn1b, wn2, blk=2000):
    grid = N // blk
    wspec = pl.BlockSpec((D, D), lambda i: (0, 0))
    rspec = pl.BlockSpec((blk, D), lambda i: (i, 0))
    return pl.pallas_call(
        _final_body,
        grid=(grid,),
        in_specs=[rspec, rspec, rspec] + [wspec] * 4,
        out_specs=rspec,
        out_shape=jax.ShapeDtypeStruct((N, D), F32),
    )(sa, sb, x, we2, wn1a, wn1b, wn2)


def _edge_pass(t_tbl, c_tbl, src_f, dst_f):
    mesh = plsc.VectorSubcoreMesh(core_axis_name="c", subcore_axis_name="s")

    @functools.partial(
        pl.kernel,
        mesh=mesh,
        out_type=(
            jax.ShapeDtypeStruct((N, D), F32),
            jax.ShapeDtypeStruct((N, D), F32),
        ),
        scratch_types=[
            pltpu.VMEM_SHARED((N, D), F32),
            pltpu.VMEM((CH, 2 * D), F32),
            pltpu.VMEM((CH, 2 * D), F32),
            pltpu.VMEM((CH, D), F32),
            pltpu.VMEM((CH,), jnp.int32),
            pltpu.VMEM((CH,), jnp.int32),
            pltpu.VMEM((CH, 2 * D), F32),
            pltpu.VMEM((CH, 2 * D), F32),
            pltpu.VMEM((CH, D), F32),
            pltpu.VMEM((CH,), jnp.int32),
            pltpu.VMEM((CH,), jnp.int32),
            pltpu.VMEM((CH, 2 * D), F32),
            pltpu.VMEM((CH, 2 * D), F32),
            pltpu.VMEM((CH, D), F32),
            pltpu.VMEM((CH,), jnp.int32),
            pltpu.VMEM((CH,), jnp.int32),
            pltpu.VMEM((CH, D), F32),
            pltpu.VMEM((CH, D), F32),
            pltpu.SemaphoreType.DMA,
            pltpu.SemaphoreType.DMA,
            pltpu.SemaphoreType.DMA,
            pltpu.SemaphoreType.DMA,
            pltpu.SemaphoreType.DMA,
            pltpu.SemaphoreType.DMA,
            pltpu.SemaphoreType.DMA,
            pltpu.SemaphoreType.DMA,
            pltpu.SemaphoreType.DMA,
            pltpu.SemaphoreType.DMA,
            pltpu.SemaphoreType.DMA,
            pltpu.SemaphoreType.DMA,
            pltpu.SemaphoreType.DMA,
            pltpu.SemaphoreType.DMA,
            pltpu.SemaphoreType.DMA,
        ],
    )
    def k(t_hbm, c_hbm, src_hbm, dst_hbm, out0, out1,
          s_sh, ts0, td0, bufc0, sidx0, didx0, ts1, td1, bufc1, sidx1, didx1,
          ts2, td2, bufc2, sidx2, didx2, of, orr,
          sem_f0, sem_r0, sem_c0, sem_si0, sem_di0,
          sem_f1, sem_r1, sem_c1, sem_si1, sem_di1,
          sem_f2, sem_r2, sem_c2, sem_si2, sem_di2):
        c = lax.axis_index("c")
        s = lax.axis_index("s")

        sets = (
            (ts0, td0, bufc0, sidx0, didx0,
             sem_f0, sem_r0, sem_c0, sem_si0, sem_di0),
            (ts1, td1, bufc1, sidx1, didx1,
             sem_f1, sem_r1, sem_c1, sem_si1, sem_di1),
            (ts2, td2, bufc2, sidx2, didx2,
             sem_f2, sem_r2, sem_c2, sem_si2, sem_di2),
        )

        def zbuf_body(r, _):
            for j in range(D // 16):
                of[r, pl.ds(j * 16, 16)] = jnp.zeros((16,), F32)
            return 0
        lax.fori_loop(0, RCH, zbuf_body, 0)

        z_lo = (NRCH * s) // NS
        z_hi = (NRCH * (s + 1)) // NS

        def zacc_body(t, _):
            pltpu.sync_copy(of, s_sh.at[pl.ds(t * RCH, RCH)])
            return 0
        lax.fori_loop(z_lo, z_hi, zacc_body, 0)
        plsc.subcore_barrier()

        base = c * EPC + s * EPW

        def fetch(i, st):
            ts, td, bufc, sidx, didx, sem_f, sem_r, sem_c, sem_si, sem_di = st
            e0 = pl.multiple_of(base + i * CH, 8)
            c_si = pltpu.async_copy(src_hbm.at[pl.ds(e0, CH)], sidx, sem_si)
            c_di = pltpu.async_copy(dst_hbm.at[pl.ds(e0, CH)], didx, sem_di)
            c_si.wait()
            c_di.wait()
            c_ts = pltpu.async_copy(t_hbm.at[sidx], ts, sem_f)
            c_td = pltpu.async_copy(t_hbm.at[didx], td, sem_r)
            c_c = pltpu.async_copy(c_hbm.at[pl.ds(e0, CH)], bufc, sem_c)
            return c_ts, c_td, c_c

        def consume(st):
            ts, td, bufc, sidx, didx, sem_f, sem_r, sem_c, sem_si, sem_di = st
            pltpu.make_async_copy(t_hbm.at[sidx], ts, sem_f).wait()
            pltpu.make_async_copy(t_hbm.at[didx], td, sem_r).wait()
            pltpu.make_async_copy(c_hbm.at[pl.ds(0, CH)], bufc, sem_c).wait()

            def vec_body(r, _):
                for j in range(D // 16):
                    lo = j * 16
                    cc = bufc[r, pl.ds(lo, 16)]
                    tf = ts[r, pl.ds(lo, 16)] + td[r, pl.ds(D + lo, 16)] + cc
                    tr = td[r, pl.ds(lo, 16)] + ts[r, pl.ds(D + lo, 16)] + cc
                    of[r, pl.ds(lo, 16)] = jnp.maximum(tf, 0.01 * tf)
                    orr[r, pl.ds(lo, 16)] = jnp.maximum(tr, 0.01 * tr)
                return 0
            lax.fori_loop(0, CH, vec_body, 0)

            pltpu.sync_copy(of, s_sh.at[didx], add=True)
            pltpu.sync_copy(orr, s_sh.at[sidx], add=True)

        fetch(0, sets[0])
        fetch(1, sets[1])

        def tri_body(p, _):
            i = 3 * p
            fetch(i + 2, sets[2])
            consume(sets[0])
            fetch(i + 3, sets[0])
            consume(sets[1])

            @pl.when(p < NTRI - 1)
            def _():
                fetch(i + 4, sets[1])
            consume(sets[2])
            return 0
        lax.fori_loop(0, NTRI, tri_body, 0)
        consume(sets[0])
        plsc.subcore_barrier()

        def rd_body(t, _):
            rows = pl.ds(t * RCH, RCH)

            @pl.when(c == 0)
            def _():
                pltpu.sync_copy(s_sh.at[rows], out0.at[rows])

            @pl.when(c == 1)
            def _():
                pltpu.sync_copy(s_sh.at[rows], out1.at[rows])
            return 0
        lax.fori_loop(z_lo, z_hi, rd_body, 0)

    return k(t_tbl, c_tbl, src_f, dst_f)


def kernel(nf, ef, edge_index, we1_0, we2_0, wn1_0, wn2_0,
           we1_1, we2_1, wn1_1, wn2_1):
    fin = nf.shape[1]
    emb = wn2_0.shape[1]
    src_f = edge_index[0].astype(jnp.int32)
    dst_f = edge_index[1].astype(jnp.int32)

    wab0 = jnp.concatenate([we1_0[:fin], we1_0[fin:2 * fin]], axis=1)
    wab1 = jnp.concatenate([we1_1[:emb], we1_1[emb:2 * emb]], axis=1)

    t0 = _projab(nf, wab0)
    c0, c1 = _projc(ef, we1_0[2 * fin:], we1_1[2 * emb:], blk=8000)

    s0a, s0b = _edge_pass(t0, c0, src_f, dst_f)

    h, t1 = _node(s0a, s0b, nf, we2_0, wn1_0[:fin], wn1_0[fin:], wn2_0, wab1)

    s1a, s1b = _edge_pass(t1, c1, src_f, dst_f)

    return _final(s1a, s1b, h, we2_1, wn1_1[:emb], wn1_1[emb:], wn2_1)


# speedup vs baseline: 1.0515x; 1.0515x over previous
import functools

import jax
import jax.numpy as jnp
from jax import lax
from jax.experimental import pallas as pl
from jax.experimental.pallas import tpu as pltpu
from jax.experimental.pallas import tpu_sc as plsc

N = 10000
E = 320000
D = 128
F32 = jnp.float32

NC = 2
NS = 16
EPC = E // NC
EPW = EPC // NS
CH = 16
NCHUNK = EPW // CH
NTRI = 208
RCH = 16
NRCH = N // RCH


def _dot(a, b):
    return jnp.dot(a, b, preferred_element_type=F32)


def _projc_body(x_ref, w1_ref, w2_ref, o1_ref, o2_ref):
    x = x_ref[...]
    o1_ref[...] = _dot(x, w1_ref[...])
    o2_ref[...] = _dot(x, w2_ref[...])


def _projc(x, w1, w2, blk):
    rows, k = x.shape
    grid = rows // blk
    ospec = pl.BlockSpec((blk, D), lambda i: (i, 0))
    return pl.pallas_call(
        _projc_body,
        grid=(grid,),
        in_specs=[
            pl.BlockSpec((blk, k), lambda i: (i, 0)),
            pl.BlockSpec((k, D), lambda i: (0, 0)),
            pl.BlockSpec((k, D), lambda i: (0, 0)),
        ],
        out_specs=[ospec, ospec],
        out_shape=[
            jax.ShapeDtypeStruct((rows, D), F32),
            jax.ShapeDtypeStruct((rows, D), F32),
        ],
    )(x, w1, w2)


def _projab_body(x_ref, w_ref, o_ref):
    o_ref[...] = _dot(x_ref[...], w_ref[...])


def _projab(x, w, blk=2000):
    grid = N // blk
    return pl.pallas_call(
        _projab_body,
        grid=(grid,),
        in_specs=[
            pl.BlockSpec((blk, D), lambda i: (i, 0)),
            pl.BlockSpec((D, 2 * D), lambda i: (0, 0)),
        ],
        out_specs=pl.BlockSpec((blk, 2 * D), lambda i: (i, 0)),
        out_shape=jax.ShapeDtypeStruct((N, 2 * D), F32),
    )(x, w)


def _red(sa_ref, sb_ref, we2_ref):
    return _dot(sa_ref[...] + sb_ref[...], we2_ref[...])


def _node_body(sa_ref, sb_ref, x_ref, we2_ref, wn1a_ref, wn1b_ref, wn2_ref,
               wab_ref, h_ref, ab_ref):
    red = _red(sa_ref, sb_ref, we2_ref)
    z = _dot(x_ref[...], wn1a_ref[...]) + _dot(red, wn1b_ref[...])
    h = _dot(jnp.maximum(z, 0.01 * z), wn2_ref[...])
    h_ref[...] = h
    ab_ref[...] = _dot(h, wab_ref[...])


def _node(sa, sb, x, we2, wn1a, wn1b, wn2, wab, blk=2000):
    grid = N // blk
    wspec = pl.BlockSpec((D, D), lambda i: (0, 0))
    rspec = pl.BlockSpec((blk, D), lambda i: (i, 0))
    return pl.pallas_call(
        _node_body,
        grid=(grid,),
        in_specs=[rspec, rspec, rspec] + [wspec] * 4
        + [pl.BlockSpec((D, 2 * D), lambda i: (0, 0))],
        out_specs=[rspec, pl.BlockSpec((blk, 2 * D), lambda i: (i, 0))],
        out_shape=[jax.ShapeDtypeStruct((N, D), F32),
                   jax.ShapeDtypeStruct((N, 2 * D), F32)],
    )(sa, sb, x, we2, wn1a, wn1b, wn2, wab)


def _final_body(sa_ref, sb_ref, x_ref, we2_ref, wn1a_ref, wn1b_ref, wn2_ref,
                o_ref):
    red = _red(sa_ref, sb_ref, we2_ref)
    z = _dot(x_ref[...], wn1a_ref[...]) + _dot(red, wn1b_ref[...])
    o_ref[...] = _dot(jnp.maximum(z, 0.01 * z), wn2_ref[...])


def _final(sa, sb, x, we2, wn1a, wn1b, wn2, blk=2000):
    grid = N // blk
    wspec = pl.BlockSpec((D, D), lambda i: (0, 0))
    rspec = pl.BlockSpec((blk, D), lambda i: (i, 0))
    return pl.pallas_call(
        _final_body,
        grid=(grid,),
        in_specs=[rspec, rspec, rspec] + [wspec] * 4,
        out_specs=rspec,
        out_shape=jax.ShapeDtypeStruct((N, D), F32),
    )(sa, sb, x, we2, wn1a, wn1b, wn2)


def _edge_pass(t_tbl, c_tbl, src_f, dst_f):
    mesh = plsc.VectorSubcoreMesh(core_axis_name="c", subcore_axis_name="s")

    @functools.partial(
        pl.kernel,
        mesh=mesh,
        out_type=(
            jax.ShapeDtypeStruct((N, D), F32),
            jax.ShapeDtypeStruct((N, D), F32),
        ),
        scratch_types=[
            pltpu.VMEM_SHARED((N, D), F32),
            pltpu.VMEM((CH, 2 * D), F32),
            pltpu.VMEM((CH, 2 * D), F32),
            pltpu.VMEM((CH, D), F32),
            pltpu.VMEM((2 * CH,), jnp.int32),
            pltpu.VMEM((CH, 2 * D), F32),
            pltpu.VMEM((CH, 2 * D), F32),
            pltpu.VMEM((CH, D), F32),
            pltpu.VMEM((2 * CH,), jnp.int32),
            pltpu.VMEM((CH, 2 * D), F32),
            pltpu.VMEM((CH, 2 * D), F32),
            pltpu.VMEM((CH, D), F32),
            pltpu.VMEM((2 * CH,), jnp.int32),
            pltpu.VMEM((2 * CH, D), F32),
            pltpu.SemaphoreType.DMA,
            pltpu.SemaphoreType.DMA,
            pltpu.SemaphoreType.DMA,
            pltpu.SemaphoreType.DMA,
            pltpu.SemaphoreType.DMA,
            pltpu.SemaphoreType.DMA,
            pltpu.SemaphoreType.DMA,
            pltpu.SemaphoreType.DMA,
            pltpu.SemaphoreType.DMA,
            pltpu.SemaphoreType.DMA,
            pltpu.SemaphoreType.DMA,
            pltpu.SemaphoreType.DMA,
            pltpu.SemaphoreType.DMA,
            pltpu.SemaphoreType.DMA,
            pltpu.SemaphoreType.DMA,
        ],
    )
    def k(t_hbm, c_hbm, src_hbm, dst_hbm, out0, out1,
          s_sh, ts0, td0, bufc0, sdidx0, ts1, td1, bufc1, sdidx1,
          ts2, td2, bufc2, sdidx2, ofr,
          sem_f0, sem_r0, sem_c0, sem_si0, sem_di0,
          sem_f1, sem_r1, sem_c1, sem_si1, sem_di1,
          sem_f2, sem_r2, sem_c2, sem_si2, sem_di2):
        c = lax.axis_index("c")
        s = lax.axis_index("s")

        sets = (
            (ts0, td0, bufc0, sdidx0,
             sem_f0, sem_r0, sem_c0, sem_si0, sem_di0),
            (ts1, td1, bufc1, sdidx1,
             sem_f1, sem_r1, sem_c1, sem_si1, sem_di1),
            (ts2, td2, bufc2, sdidx2,
             sem_f2, sem_r2, sem_c2, sem_si2, sem_di2),
        )

        def zbuf_body(r, _):
            for j in range(D // 16):
                ofr[r, pl.ds(j * 16, 16)] = jnp.zeros((16,), F32)
            return 0
        lax.fori_loop(0, RCH, zbuf_body, 0)

        z_lo = (NRCH * s) // NS
        z_hi = (NRCH * (s + 1)) // NS

        def zacc_body(t, _):
            pltpu.sync_copy(ofr.at[pl.ds(0, RCH)], s_sh.at[pl.ds(t * RCH, RCH)])
            return 0
        lax.fori_loop(z_lo, z_hi, zacc_body, 0)
        plsc.subcore_barrier()

        base = c * EPC + s * EPW

        def fetch(i, st):
            ts, td, bufc, sdidx, sem_f, sem_r, sem_c, sem_si, sem_di = st
            e0 = pl.multiple_of(base + i * CH, 8)
            c_di = pltpu.async_copy(dst_hbm.at[pl.ds(e0, CH)],
                                    sdidx.at[pl.ds(0, CH)], sem_di)
            c_si = pltpu.async_copy(src_hbm.at[pl.ds(e0, CH)],
                                    sdidx.at[pl.ds(CH, CH)], sem_si)
            c_di.wait()
            c_si.wait()
            c_ts = pltpu.async_copy(t_hbm.at[sdidx.at[pl.ds(CH, CH)]], ts,
                                    sem_f)
            c_td = pltpu.async_copy(t_hbm.at[sdidx.at[pl.ds(0, CH)]], td,
                                    sem_r)
            c_c = pltpu.async_copy(c_hbm.at[pl.ds(e0, CH)], bufc, sem_c)
            return c_ts, c_td, c_c

        def consume(st):
            ts, td, bufc, sdidx, sem_f, sem_r, sem_c, sem_si, sem_di = st
            pltpu.make_async_copy(t_hbm.at[sdidx.at[pl.ds(CH, CH)]], ts,
                                  sem_f).wait()
            pltpu.make_async_copy(t_hbm.at[sdidx.at[pl.ds(0, CH)]], td,
                                  sem_r).wait()
            pltpu.make_async_copy(c_hbm.at[pl.ds(0, CH)], bufc, sem_c).wait()

            def vec_body(r, _):
                for j in range(D // 16):
                    lo = j * 16
                    cc = bufc[r, pl.ds(lo, 16)]
                    tf = ts[r, pl.ds(lo, 16)] + td[r, pl.ds(D + lo, 16)] + cc
                    tr = td[r, pl.ds(lo, 16)] + ts[r, pl.ds(D + lo, 16)] + cc
                    ofr[r, pl.ds(lo, 16)] = jnp.maximum(tf, 0.01 * tf)
                    ofr[CH + r, pl.ds(lo, 16)] = jnp.maximum(tr, 0.01 * tr)
                return 0
            lax.fori_loop(0, CH, vec_body, 0)

            pltpu.sync_copy(ofr, s_sh.at[sdidx], add=True)

        fetch(0, sets[0])
        fetch(1, sets[1])

        def tri_body(p, _):
            i = 3 * p
            fetch(i + 2, sets[2])
            consume(sets[0])
            fetch(i + 3, sets[0])
            consume(sets[1])

            @pl.when(p < NTRI - 1)
            def _():
                fetch(i + 4, sets[1])
            consume(sets[2])
            return 0
        lax.fori_loop(0, NTRI, tri_body, 0)
        consume(sets[0])
        plsc.subcore_barrier()

        def rd_body(t, _):
            rows = pl.ds(t * RCH, RCH)

            @pl.when(c == 0)
            def _():
                pltpu.sync_copy(s_sh.at[rows], out0.at[rows])

            @pl.when(c == 1)
            def _():
                pltpu.sync_copy(s_sh.at[rows], out1.at[rows])
            return 0
        lax.fori_loop(z_lo, z_hi, rd_body, 0)

    return k(t_tbl, c_tbl, src_f, dst_f)


def kernel(nf, ef, edge_index, we1_0, we2_0, wn1_0, wn2_0,
           we1_1, we2_1, wn1_1, wn2_1):
    fin = nf.shape[1]
    emb = wn2_0.shape[1]
    src_f = edge_index[0].astype(jnp.int32)
    dst_f = edge_index[1].astype(jnp.int32)

    wab0 = jnp.concatenate([we1_0[:fin], we1_0[fin:2 * fin]], axis=1)
    wab1 = jnp.concatenate([we1_1[:emb], we1_1[emb:2 * emb]], axis=1)

    t0 = _projab(nf, wab0)
    c0, c1 = _projc(ef, we1_0[2 * fin:], we1_1[2 * emb:], blk=8000)

    s0a, s0b = _edge_pass(t0, c0, src_f, dst_f)

    h, t1 = _node(s0a, s0b, nf, we2_0, wn1_0[:fin], wn1_0[fin:], wn2_0, wab1)

    s1a, s1b = _edge_pass(t1, c1, src_f, dst_f)

    return _final(s1a, s1b, h, we2_1, wn1_1[:emb], wn1_1[emb:], wn2_1)
